# baseline (speedup 1.0000x reference)
import functools

import jax
import jax.numpy as jnp
from jax import lax
from jax.experimental import pallas as pl
from jax.experimental.pallas import tpu as pltpu
from jax.experimental.pallas import tpu_sc as plsc

_BATCH = 16384
_D = 64
_NW = 32
_RPT = _BATCH // _NW
_CHUNK = 128
_NCHUNK = _RPT // _CHUNK

_MARGIN = 1.0
_EPS2 = 0.1 * 0.1
_C = 0.25

_N_ENT = 1000000
_EBLK = 40960


def _scan_repack_body(entt_ref, relt_ref, normt_ref, entp_ref, out_ref,
                      acc_ref):
    i = pl.program_id(0)

    @pl.when(i == 0)
    def _init():
        acc_ref[0] = jnp.float32(0.0)

    blk = entt_ref[...]
    ss = jnp.sum(blk * blk, axis=0, keepdims=True)
    col = i * _EBLK + jax.lax.broadcasted_iota(jnp.int32, (1, _EBLK), 1)
    norms = jnp.where(col < _N_ENT, jnp.sqrt(ss), jnp.float32(0.0))
    acc_ref[0] += jnp.sum(norms)

    eye = jnp.eye(_D, dtype=jnp.float32)
    t = jax.lax.dot_general(blk, eye, (((0,), (0,)), ((), ())),
                            preferred_element_type=jnp.float32)
    entp_ref[:, 0:_D] = t[0:_EBLK // 2]
    entp_ref[:, _D:2 * _D] = t[_EBLK // 2:_EBLK]

    @pl.when(i == pl.num_programs(0) - 1)
    def _final():
        rel = relt_ref[...]
        nrm = normt_ref[...]
        rn = jnp.sum(jnp.sum(nrm, axis=0) * jnp.sum(rel, axis=0))
        den = jnp.sum(jnp.sqrt(jnp.sum(rel * rel, axis=0)))
        rel_loss = jnp.maximum(rn / den - jnp.float32(_EPS2), jnp.float32(0.0))
        ent_loss = jnp.maximum(acc_ref[0] - jnp.float32(1.0), jnp.float32(0.0))
        out_ref[0, 0] = jnp.float32(_C) * (ent_loss + rel_loss)


@jax.jit
def _tc_scan_repack(entt, relt, normt):
    n_ent = entt.shape[1]
    grid = ((n_ent + _EBLK - 1) // _EBLK,)
    return pl.pallas_call(
        _scan_repack_body,
        grid=grid,
        in_specs=[
            pl.BlockSpec((_D, _EBLK), lambda i: (0, i)),
            pl.BlockSpec((_D, relt.shape[1]), lambda i: (0, 0)),
            pl.BlockSpec((_D, normt.shape[1]), lambda i: (0, 0)),
        ],
        out_specs=[
            pl.BlockSpec((_EBLK // 2, 2 * _D), lambda i: (i, 0)),
            pl.BlockSpec(memory_space=pltpu.SMEM),
        ],
        out_shape=[
            jax.ShapeDtypeStruct((grid[0] * (_EBLK // 2), 2 * _D),
                                 jnp.float32),
            jax.ShapeDtypeStruct((1, 1), jnp.float32),
        ],
        scratch_shapes=[pltpu.SMEM((1,), jnp.float32)],
    )(entt, relt, normt)


def _row_hinge(bufs, r, oh, ot, ojh, ojt, lane):
    rv = jnp.broadcast_to(r, (16,))

    def side(h_buf, t_buf, rn_buf, off_h, off_t):
        dot_v = jnp.zeros((16,), jnp.float32)
        ss_v = jnp.zeros((16,), jnp.float32)
        a = []
        n = []
        for j in range(_D // 16):
            base = lane + j * 16
            h = plsc.load_gather(h_buf, [rv, off_h + base])
            t = plsc.load_gather(t_buf, [rv, off_t + base])
            rr = rn_buf[r, pl.ds(j * 16, 16)]
            nn = rn_buf[r, pl.ds(_D + j * 16, 16)]
            a.append(h + rr - t)
            n.append(nn)
            dot_v = dot_v + (h - t) * nn
            ss_v = ss_v + nn * nn
        dot = jnp.broadcast_to(jnp.sum(dot_v), (16,))
        ss = jnp.broadcast_to(jnp.sum(ss_v), (16,))
        f = dot / (ss + jnp.float32(1e-24))
        acc_v = jnp.abs(a[0] - f * n[0])
        for j in range(1, _D // 16):
            acc_v = acc_v + jnp.abs(a[j] - f * n[j])
        return jnp.sum(acc_v)

    ph, pt, prn, nh, nt, nrn = bufs
    pos = side(ph, pt, prn, oh, ot)
    neg = side(nh, nt, nrn, ojh, ojt)
    return jnp.maximum(pos - neg + jnp.float32(_MARGIN), jnp.float32(0.0))


def _sc_body(kh_hbm, kt_hbm, ir_hbm, kjh_hbm, kjt_hbm, jr_hbm, w_hbm,
             entp_hbm, relnorm_hbm, out_hbm,
             kh_v, kt_v, ir_v, kjh_v, kjt_v, jr_v, w_v,
             b_ph, b_pt, b_prn, b_nh, b_nt, b_nrn,
             out_v, sem):
    wid = lax.axis_index("s") * 2 + lax.axis_index("c")
    base = wid * _RPT

    pltpu.sync_copy(kh_hbm.at[pl.ds(base, _RPT)], kh_v)
    pltpu.sync_copy(kt_hbm.at[pl.ds(base, _RPT)], kt_v)
    pltpu.sync_copy(ir_hbm.at[pl.ds(base, _RPT)], ir_v)
    pltpu.sync_copy(kjh_hbm.at[pl.ds(base, _RPT)], kjh_v)
    pltpu.sync_copy(kjt_hbm.at[pl.ds(base, _RPT)], kjt_v)
    pltpu.sync_copy(jr_hbm.at[pl.ds(base, _RPT)], jr_v)
    pltpu.sync_copy(w_hbm.at[pl.ds(base, _RPT)], w_v)

    bufs = (b_ph, b_pt, b_prn, b_nh, b_nt, b_nrn)
    lane = lax.iota(jnp.int32, 16)
    acc = jnp.float32(0.0)
    for k in range(_NCHUNK):
        sl = pl.ds(k * _CHUNK, _CHUNK)
        copies = [
            pltpu.async_copy(entp_hbm.at[kh_v.at[sl]], b_ph, sem),
            pltpu.async_copy(entp_hbm.at[kt_v.at[sl]], b_pt, sem),
            pltpu.async_copy(relnorm_hbm.at[ir_v.at[sl]], b_prn, sem),
            pltpu.async_copy(entp_hbm.at[kjh_v.at[sl]], b_nh, sem),
            pltpu.async_copy(entp_hbm.at[kjt_v.at[sl]], b_nt, sem),
            pltpu.async_copy(relnorm_hbm.at[jr_v.at[sl]], b_nrn, sem),
        ]
        for c in copies:
            c.wait()

        def group_body(g, a):
            wv = w_v[pl.ds(k * _CHUNK + g * 16, 16)]
            for ri in range(16):
                r = g * 16 + ri
                wr = jnp.broadcast_to(wv[ri], (16,))
                oh = (wr & 1) * _D
                ot = ((wr >> 1) & 1) * _D
                ojh = ((wr >> 2) & 1) * _D
                ojt = ((wr >> 3) & 1) * _D
                a = a + _row_hinge(bufs, r, oh, ot, ojh, ojt, lane)
            return a

        acc = lax.fori_loop(0, _CHUNK // 16, group_body, acc)

    out_v[...] = jnp.where(lane == 0, acc, jnp.float32(0.0))
    pltpu.sync_copy(out_v, out_hbm.at[pl.ds(wid * 16, 16)])


@jax.jit
def _sc_hinge(kh, kt, ir, kjh, kjt, jr, w, entp, relnorm):
    mesh = plsc.VectorSubcoreMesh(core_axis_name="c", subcore_axis_name="s")
    f = pl.kernel(
        _sc_body,
        mesh=mesh,
        compiler_params=pltpu.CompilerParams(needs_layout_passes=False),
        out_type=jax.ShapeDtypeStruct((_NW * 16,), jnp.float32),
        scratch_types=[
            pltpu.VMEM((_RPT,), jnp.int32),
            pltpu.VMEM((_RPT,), jnp.int32),
            pltpu.VMEM((_RPT,), jnp.int32),
            pltpu.VMEM((_RPT,), jnp.int32),
            pltpu.VMEM((_RPT,), jnp.int32),
            pltpu.VMEM((_RPT,), jnp.int32),
            pltpu.VMEM((_RPT,), jnp.int32),
            pltpu.VMEM((_CHUNK, 2 * _D), jnp.float32),
            pltpu.VMEM((_CHUNK, 2 * _D), jnp.float32),
            pltpu.VMEM((_CHUNK, 2 * _D), jnp.float32),
            pltpu.VMEM((_CHUNK, 2 * _D), jnp.float32),
            pltpu.VMEM((_CHUNK, 2 * _D), jnp.float32),
            pltpu.VMEM((_CHUNK, 2 * _D), jnp.float32),
            pltpu.VMEM((16,), jnp.float32),
            pltpu.SemaphoreType.DMA,
        ],
    )
    return f(kh, kt, ir, kjh, kjt, jr, w, entp, relnorm)


def kernel(x, ent_table, rel_table, norm_table):
    entp, reg = _tc_scan_repack(ent_table.T, rel_table.T, norm_table.T)
    relnorm = jnp.concatenate([rel_table, norm_table], axis=1)
    ih, it, ir = x[:, 0], x[:, 1], x[:, 2]
    jh, jt, jr = x[:, 3], x[:, 4], x[:, 5]

    def line(e):
        return (e // _EBLK) * (_EBLK // 2) + (e % _EBLK) % (_EBLK // 2)

    def half(e):
        return (e % _EBLK) // (_EBLK // 2)

    w = (half(ih) | (half(it) << 1) | (half(jh) << 2) | (half(jt) << 3))
    hinge_parts = _sc_hinge(line(ih), line(it), ir, line(jh), line(jt), jr,
                            w, entp, relnorm)
    return jnp.sum(hinge_parts) + reg[0, 0]


# speedup vs baseline: 1.0772x; 1.0772x over previous
import functools

import jax
import jax.numpy as jnp
from jax import lax
from jax.experimental import pallas as pl
from jax.experimental.pallas import tpu as pltpu
from jax.experimental.pallas import tpu_sc as plsc

_BATCH = 16384
_D = 64
_NW = 32
_RPT = _BATCH // _NW
_CHUNK = 64
_NCHUNK = _RPT // _CHUNK

_MARGIN = 1.0
_EPS2 = 0.1 * 0.1
_C = 0.25

_N_ENT = 1000000
_EBLK = 32768


def _scan_repack_body(entt_ref, relt_ref, normt_ref, entp_ref, out_ref,
                      acc_ref):
    i = pl.program_id(0)

    @pl.when(i == 0)
    def _init():
        acc_ref[0] = jnp.float32(0.0)

    blk = entt_ref[...]
    ss = jnp.sum(blk * blk, axis=0, keepdims=True)
    col = i * _EBLK + jax.lax.broadcasted_iota(jnp.int32, (1, _EBLK), 1)
    norms = jnp.where(col < _N_ENT, jnp.sqrt(ss), jnp.float32(0.0))
    acc_ref[0] += jnp.sum(norms)

    eye = jnp.eye(_D, dtype=jnp.float32)
    t = jax.lax.dot_general(blk, eye, (((0,), (0,)), ((), ())),
                            preferred_element_type=jnp.float32)
    entp_ref[:, 0:_D] = t[0:_EBLK // 2]
    entp_ref[:, _D:2 * _D] = t[_EBLK // 2:_EBLK]

    @pl.when(i == pl.num_programs(0) - 1)
    def _final():
        rel = relt_ref[...]
        nrm = normt_ref[...]
        rn = jnp.sum(jnp.sum(nrm, axis=0) * jnp.sum(rel, axis=0))
        den = jnp.sum(jnp.sqrt(jnp.sum(rel * rel, axis=0)))
        rel_loss = jnp.maximum(rn / den - jnp.float32(_EPS2), jnp.float32(0.0))
        ent_loss = jnp.maximum(acc_ref[0] - jnp.float32(1.0), jnp.float32(0.0))
        out_ref[0, 0] = jnp.float32(_C) * (ent_loss + rel_loss)


@jax.jit
def _tc_scan_repack(entt, relt, normt):
    n_ent = entt.shape[1]
    grid = ((n_ent + _EBLK - 1) // _EBLK,)
    return pl.pallas_call(
        _scan_repack_body,
        grid=grid,
        in_specs=[
            pl.BlockSpec((_D, _EBLK), lambda i: (0, i)),
            pl.BlockSpec((_D, relt.shape[1]), lambda i: (0, 0)),
            pl.BlockSpec((_D, normt.shape[1]), lambda i: (0, 0)),
        ],
        out_specs=[
            pl.BlockSpec((_EBLK // 2, 2 * _D), lambda i: (i, 0)),
            pl.BlockSpec(memory_space=pltpu.SMEM),
        ],
        out_shape=[
            jax.ShapeDtypeStruct((grid[0] * (_EBLK // 2), 2 * _D),
                                 jnp.float32),
            jax.ShapeDtypeStruct((1, 1), jnp.float32),
        ],
        scratch_shapes=[pltpu.SMEM((1,), jnp.float32)],
    )(entt, relt, normt)


def _row_hinge(bufs, r, oh, ot, ojh, ojt, lane):
    rv = jnp.broadcast_to(r, (16,))

    def side(h_buf, t_buf, rn_buf, off_h, off_t):
        dot_v = jnp.zeros((16,), jnp.float32)
        ss_v = jnp.zeros((16,), jnp.float32)
        a = []
        n = []
        for j in range(_D // 16):
            base = lane + j * 16
            h = plsc.load_gather(h_buf, [rv, off_h + base])
            t = plsc.load_gather(t_buf, [rv, off_t + base])
            rr = rn_buf[r, pl.ds(j * 16, 16)]
            nn = rn_buf[r, pl.ds(_D + j * 16, 16)]
            a.append(h + rr - t)
            n.append(nn)
            dot_v = dot_v + (h - t) * nn
            ss_v = ss_v + nn * nn
        dot = jnp.broadcast_to(jnp.sum(dot_v), (16,))
        ss = jnp.broadcast_to(jnp.sum(ss_v), (16,))
        f = dot / (ss + jnp.float32(1e-24))
        acc_v = jnp.abs(a[0] - f * n[0])
        for j in range(1, _D // 16):
            acc_v = acc_v + jnp.abs(a[j] - f * n[j])
        return jnp.sum(acc_v)

    ph, pt, prn, nh, nt, nrn = bufs
    pos = side(ph, pt, prn, oh, ot)
    neg = side(nh, nt, nrn, ojh, ojt)
    return jnp.maximum(pos - neg + jnp.float32(_MARGIN), jnp.float32(0.0))


def _sc_body(kh_hbm, kt_hbm, ir_hbm, kjh_hbm, kjt_hbm, jr_hbm, w_hbm,
             entp_hbm, relnorm_hbm, out_hbm,
             kh_v, kt_v, ir_v, kjh_v, kjt_v, jr_v, w_v,
             b_ph0, b_pt0, b_prn0, b_nh0, b_nt0, b_nrn0,
             b_ph1, b_pt1, b_prn1, b_nh1, b_nt1, b_nrn1,
             out_v, sem0, sem1):
    wid = lax.axis_index("s") * 2 + lax.axis_index("c")
    base = wid * _RPT

    pltpu.sync_copy(kh_hbm.at[pl.ds(base, _RPT)], kh_v)
    pltpu.sync_copy(kt_hbm.at[pl.ds(base, _RPT)], kt_v)
    pltpu.sync_copy(ir_hbm.at[pl.ds(base, _RPT)], ir_v)
    pltpu.sync_copy(kjh_hbm.at[pl.ds(base, _RPT)], kjh_v)
    pltpu.sync_copy(kjt_hbm.at[pl.ds(base, _RPT)], kjt_v)
    pltpu.sync_copy(jr_hbm.at[pl.ds(base, _RPT)], jr_v)
    pltpu.sync_copy(w_hbm.at[pl.ds(base, _RPT)], w_v.at[pl.ds(0, _RPT)])

    bufsets = (
        (b_ph0, b_pt0, b_prn0, b_nh0, b_nt0, b_nrn0),
        (b_ph1, b_pt1, b_prn1, b_nh1, b_nt1, b_nrn1),
    )
    sems = (sem0, sem1)
    lane = lax.iota(jnp.int32, 16)

    def fire(k, s):
        bp, bt, brn, bnh, bnt, bnrn = bufsets[s]
        sl = pl.ds(k * _CHUNK, _CHUNK)
        return [
            pltpu.async_copy(entp_hbm.at[kh_v.at[sl]], bp, sems[s]),
            pltpu.async_copy(entp_hbm.at[kt_v.at[sl]], bt, sems[s]),
            pltpu.async_copy(relnorm_hbm.at[ir_v.at[sl]], brn, sems[s]),
            pltpu.async_copy(entp_hbm.at[kjh_v.at[sl]], bnh, sems[s]),
            pltpu.async_copy(entp_hbm.at[kjt_v.at[sl]], bnt, sems[s]),
            pltpu.async_copy(relnorm_hbm.at[jr_v.at[sl]], bnrn, sems[s]),
        ]

    acc = jnp.float32(0.0)
    pending = fire(0, 0)
    for k in range(_NCHUNK):
        cur = pending
        if k + 1 < _NCHUNK:
            pending = fire(k + 1, (k + 1) % 2)
        for c in cur:
            c.wait()
        bufs = bufsets[k % 2]

        def group_body(g, a, k=k, bufs=bufs):
            wv = w_v[pl.ds(k * _CHUNK + g * 8, 16)]
            for ri in range(8):
                r = g * 8 + ri
                wr = jnp.broadcast_to(wv[ri], (16,))
                oh = (wr & 1) * _D
                ot = ((wr >> 1) & 1) * _D
                ojh = ((wr >> 2) & 1) * _D
                ojt = ((wr >> 3) & 1) * _D
                a = a + _row_hinge(bufs, r, oh, ot, ojh, ojt, lane)
            return a

        acc = lax.fori_loop(0, _CHUNK // 8, group_body, acc)

    out_v[...] = jnp.where(lane == 0, acc, jnp.float32(0.0))
    pltpu.sync_copy(out_v, out_hbm.at[pl.ds(wid * 16, 16)])


@jax.jit
def _sc_hinge(kh, kt, ir, kjh, kjt, jr, w, entp, relnorm):
    mesh = plsc.VectorSubcoreMesh(core_axis_name="c", subcore_axis_name="s")
    f = pl.kernel(
        _sc_body,
        mesh=mesh,
        compiler_params=pltpu.CompilerParams(needs_layout_passes=False),
        out_type=jax.ShapeDtypeStruct((_NW * 16,), jnp.float32),
        scratch_types=(
            [pltpu.VMEM((_RPT,), jnp.int32)] * 6
            + [pltpu.VMEM((_RPT + 16,), jnp.int32)]
            + [pltpu.VMEM((_CHUNK, 2 * _D), jnp.float32)] * 12
            + [pltpu.VMEM((16,), jnp.float32),
               pltpu.SemaphoreType.DMA,
               pltpu.SemaphoreType.DMA]
        ),
    )
    return f(kh, kt, ir, kjh, kjt, jr, w, entp, relnorm)


def kernel(x, ent_table, rel_table, norm_table):
    entp, reg = _tc_scan_repack(ent_table.T, rel_table.T, norm_table.T)
    relnorm = jnp.concatenate([rel_table, norm_table], axis=1)
    ih, it, ir = x[:, 0], x[:, 1], x[:, 2]
    jh, jt, jr = x[:, 3], x[:, 4], x[:, 5]

    def line(e):
        return (e // _EBLK) * (_EBLK // 2) + (e % _EBLK) % (_EBLK // 2)

    def half(e):
        return (e % _EBLK) // (_EBLK // 2)

    w = (half(ih) | (half(it) << 1) | (half(jh) << 2) | (half(jt) << 3))
    hinge_parts = _sc_hinge(line(ih), line(it), ir, line(jh), line(jt), jr,
                            w, entp, relnorm)
    return jnp.sum(hinge_parts) + reg[0, 0]
